# Initial kernel scaffold; baseline (speedup 1.0000x reference)
import jax
import jax.numpy as jnp
from jax.experimental import pallas as pl


def kernel(boxes, scores):
    raise NotImplementedError("write your pallas kernel here")


# speedup vs baseline: 21260.4124x; 21260.4124x over previous
import jax
import jax.numpy as jnp
import numpy as np
from jax import lax
from jax.experimental import pallas as pl
from jax.experimental.pallas import tpu as pltpu
from jax.experimental.pallas import tpu_sc as plsc

B = 4
N = 5000
TOPK = 5
L = 16
NPAD = 5120
NCH = NPAD // L
OUTW = 128
SUP = 64
NO = 80
CONF = np.float32(0.2)
IOU = np.float32(0.4)
NEG = np.float32(-1e30)
NEGHALF = np.float32(-5e29)
EPS = np.float32(1e-9)
BIGI = np.int32(2**30)


def _nms_body(bt_hbm, sp_hbm, out_hbm,
              x1_v, y1_v, x2_v, y2_v, ws_v, area_v, out_v, si, sf):
    cid = lax.axis_index("c")
    sid = lax.axis_index("s")
    wid = sid * 2 + cid

    @pl.when(wid < B)
    def _():
        b = wid
        pltpu.sync_copy(bt_hbm.at[b, 0], x1_v.at[pl.ds(0, NPAD)])
        pltpu.sync_copy(bt_hbm.at[b, 1], y1_v.at[pl.ds(0, NPAD)])
        pltpu.sync_copy(bt_hbm.at[b, 2], x2_v.at[pl.ds(0, NPAD)])
        pltpu.sync_copy(bt_hbm.at[b, 3], y2_v.at[pl.ds(0, NPAD)])
        pltpu.sync_copy(sp_hbm.at[b], ws_v)

        iota = lax.iota(jnp.int32, L)
        zero16 = jnp.zeros((L,), jnp.float32)

        def pa(i, m16):
            return jnp.maximum(m16, ws_v[pl.ds(i * L, L)])
        m16 = lax.fori_loop(0, NCH, pa, jnp.full((L,), NEG, jnp.float32))
        for s in (8, 4, 2, 1):
            m16 = jnp.maximum(m16, m16[iota ^ s])
        thr = jnp.where(m16[0] > CONF, CONF, np.float32(0.0))

        def pb(i, _):
            sl = pl.ds(i * L, L)
            s = ws_v[sl]
            area_v[sl] = (x2_v[sl] - x1_v[sl]) * (y2_v[sl] - y1_v[sl])
            ws_v[sl] = jnp.where(s > thr, s, NEG)
            return 0
        lax.fori_loop(0, NCH, pb, 0)

        def pz(r, _):
            out_v[pl.ds(r * L, L)] = zero16
            return 0
        lax.fori_loop(0, OUTW // L, pz, 0)

        si[0] = np.int32(0)
        si[1] = np.int32(1)
        si[2] = np.int32(-1)
        sf[0] = np.float32(0.0)
        sf[1] = np.float32(0.0)
        sf[2] = np.float32(0.0)
        sf[3] = np.float32(0.0)
        sf[4] = np.float32(0.0)

        def nms_step():
            prev_idx = si[2]
            px1 = sf[0]
            py1 = sf[1]
            px2 = sf[2]
            py2 = sf[3]
            par = sf[4]

            def step(i, carry):
                bv, bi = carry
                base = i * L
                sl = pl.ds(base, L)
                ws = ws_v[sl]
                x1 = x1_v[sl]
                y1 = y1_v[sl]
                x2 = x2_v[sl]
                y2 = y2_v[sl]
                ar = area_v[sl]
                ix1 = jnp.maximum(px1, x1)
                iy1 = jnp.maximum(py1, y1)
                ix2 = jnp.minimum(px2, x2)
                iy2 = jnp.minimum(py2, y2)
                inter = (jnp.maximum(ix2 - ix1, 0.0)
                         * jnp.maximum(iy2 - iy1, 0.0))
                iou = inter / (par + ar - inter + EPS)
                idxv = iota + base
                sup = (iou > IOU) | (idxv == prev_idx)
                wsn = jnp.where(sup, NEG, ws)
                ws_v[sl] = wsn
                better = (wsn > bv) | ((wsn == bv) & (idxv < bi))
                bv = jnp.where(better, wsn, bv)
                bi = jnp.where(better, idxv, bi)
                return bv, bi

            bv, bi = lax.fori_loop(
                0, NCH, step,
                (jnp.full((L,), NEG, jnp.float32), jnp.full((L,), BIGI)))

            for s in (8, 4, 2, 1):
                gv = bv[iota ^ s]
                gi = bi[iota ^ s]
                better = (gv > bv) | ((gv == bv) & (gi < bi))
                bv = jnp.where(better, gv, bv)
                bi = jnp.where(better, gi, bi)
            m = bv[0]
            sel = bi[0]
            have = m > NEGHALF

            cx1 = x1_v[pl.ds(sel, L)][0]
            cy1 = y1_v[pl.ds(sel, L)][0]
            cx2 = x2_v[pl.ds(sel, L)][0]
            cy2 = y2_v[pl.ds(sel, L)][0]
            car = area_v[pl.ds(sel, L)][0]
            nontiny = (cx2 - cx1 >= 1.0) & (cy2 - cy1 >= 1.0)

            @pl.when(have & nontiny)
            def _():
                row = jnp.where(iota == 0, cx1,
                                jnp.where(iota == 1, cy1,
                                          jnp.where(iota == 2, cx2,
                                                    jnp.where(iota == 3, cy2,
                                                              m))))
                out_v[pl.ds(si[0] * L, L)] = row
                si[0] = si[0] + np.int32(1)

            si[1] = jnp.where(have & (si[0] < TOPK),
                              np.int32(1), np.int32(0))
            si[2] = sel
            sf[0] = cx1
            sf[1] = cy1
            sf[2] = cx2
            sf[3] = cy2
            sf[4] = car

        def outer(o, _):
            @pl.when(si[1] == np.int32(1))
            def _():
                def inner(k, __):
                    @pl.when(si[1] == np.int32(1))
                    def _():
                        nms_step()
                    return 0
                lax.fori_loop(0, SUP, inner, 0)
            return 0
        lax.fori_loop(0, NO, outer, 0)

        pltpu.sync_copy(out_v, out_hbm.at[b])


@jax.jit
def _nms_call(bt, sp):
    mesh = plsc.VectorSubcoreMesh(core_axis_name="c", subcore_axis_name="s")
    f = pl.kernel(
        _nms_body,
        out_type=jax.ShapeDtypeStruct((B, OUTW), jnp.float32),
        mesh=mesh,
        scratch_types=[
            pltpu.VMEM((NPAD + L,), jnp.float32),
            pltpu.VMEM((NPAD + L,), jnp.float32),
            pltpu.VMEM((NPAD + L,), jnp.float32),
            pltpu.VMEM((NPAD + L,), jnp.float32),
            pltpu.VMEM((NPAD,), jnp.float32),
            pltpu.VMEM((NPAD + L,), jnp.float32),
            pltpu.VMEM((OUTW,), jnp.float32),
            pltpu.SMEM((4,), jnp.int32),
            pltpu.SMEM((8,), jnp.float32),
        ],
    )
    return f(bt, sp)


def kernel(boxes, scores):
    bt = jnp.transpose(boxes, (0, 2, 1))
    bt = jnp.pad(bt, ((0, 0), (0, 0), (0, NPAD - N)))
    sp = jnp.pad(scores, ((0, 0), (0, NPAD - N)))
    out = _nms_call(bt, sp)
    return out[:, :TOPK * L].reshape(B, TOPK, L)[:, :, :TOPK]


# speedup vs baseline: 23704.5263x; 1.1150x over previous
import jax
import jax.numpy as jnp
import numpy as np
from jax import lax
from jax.experimental import pallas as pl
from jax.experimental.pallas import tpu as pltpu
from jax.experimental.pallas import tpu_sc as plsc

B = 4
N = 5000
TOPK = 5
L = 16
NPAD = 5120
NCH = NPAD // L
OUTW = 128
SUP = 64
NO = 80
UNROLL = 4
CONF = np.float32(0.2)
IOU = np.float32(0.4)
NEG = np.float32(-1e30)
NEGHALF = np.float32(-5e29)
EPS = np.float32(1e-9)
BIGI = np.int32(2**30)


def _nms_body(bt_hbm, sp_hbm, out_hbm,
              x1_v, y1_v, x2_v, y2_v, ws_v, area_v, out_v, si, sf):
    cid = lax.axis_index("c")
    sid = lax.axis_index("s")
    wid = sid * 2 + cid

    @pl.when(wid < B)
    def _():
        b = wid
        pltpu.sync_copy(bt_hbm.at[b, 0], x1_v.at[pl.ds(0, NPAD)])
        pltpu.sync_copy(bt_hbm.at[b, 1], y1_v.at[pl.ds(0, NPAD)])
        pltpu.sync_copy(bt_hbm.at[b, 2], x2_v.at[pl.ds(0, NPAD)])
        pltpu.sync_copy(bt_hbm.at[b, 3], y2_v.at[pl.ds(0, NPAD)])
        pltpu.sync_copy(sp_hbm.at[b], ws_v)

        iota = lax.iota(jnp.int32, L)
        zero16 = jnp.zeros((L,), jnp.float32)

        @plsc.parallel_loop(0, NCH, step=1, unroll=UNROLL,
                            carry=jnp.full((L,), NEG, jnp.float32))
        def m16(i, m):
            sl = pl.ds(i * L, L)
            area_v[sl] = (x2_v[sl] - x1_v[sl]) * (y2_v[sl] - y1_v[sl])
            return jnp.maximum(m, ws_v[sl])
        mr = m16
        for s in (8, 4, 2, 1):
            mr = jnp.maximum(mr, mr[iota ^ s])
        thr = jnp.where(mr[0] > CONF, CONF, np.float32(0.0))

        def pz(r, _):
            out_v[pl.ds(r * L, L)] = zero16
            return 0
        lax.fori_loop(0, OUTW // L, pz, 0)

        si[0] = np.int32(0)
        si[1] = np.int32(1)
        si[2] = np.int32(-1)

        def nms_step(first):
            if first:
                prev_idx = np.int32(-1)
            else:
                prev_idx = si[2]
                px1 = sf[0]
                py1 = sf[1]
                px2 = sf[2]
                py2 = sf[3]
                par = sf[4]

            @plsc.parallel_loop(
                0, NCH, step=1, unroll=UNROLL,
                carry=(jnp.full((L,), NEG, jnp.float32),
                       jnp.full((L,), BIGI)))
            def final(i, carry):
                bv, bi = carry
                base = i * L
                sl = pl.ds(base, L)
                ws = ws_v[sl]
                idxv = iota + base
                if first:
                    wsn = jnp.where(ws > thr, ws, NEG)
                else:
                    x1 = x1_v[sl]
                    y1 = y1_v[sl]
                    x2 = x2_v[sl]
                    y2 = y2_v[sl]
                    ar = area_v[sl]
                    ix1 = jnp.maximum(px1, x1)
                    iy1 = jnp.maximum(py1, y1)
                    ix2 = jnp.minimum(px2, x2)
                    iy2 = jnp.minimum(py2, y2)
                    inter = (jnp.maximum(ix2 - ix1, 0.0)
                             * jnp.maximum(iy2 - iy1, 0.0))
                    iou = inter / (par + ar - inter + EPS)
                    sup = (iou > IOU) | (idxv == prev_idx)
                    wsn = jnp.where(sup, NEG, ws)
                ws_v[sl] = wsn
                better = (wsn > bv) | ((wsn == bv) & (idxv < bi))
                bv = jnp.where(better, wsn, bv)
                bi = jnp.where(better, idxv, bi)
                return bv, bi

            bv, bi = final

            for s in (8, 4, 2, 1):
                gv = bv[iota ^ s]
                gi = bi[iota ^ s]
                better = (gv > bv) | ((gv == bv) & (gi < bi))
                bv = jnp.where(better, gv, bv)
                bi = jnp.where(better, gi, bi)
            m = bv[0]
            sel = bi[0]
            have = m > NEGHALF

            cx1 = x1_v[pl.ds(sel, L)][0]
            cy1 = y1_v[pl.ds(sel, L)][0]
            cx2 = x2_v[pl.ds(sel, L)][0]
            cy2 = y2_v[pl.ds(sel, L)][0]
            car = area_v[pl.ds(sel, L)][0]
            nontiny = (cx2 - cx1 >= 1.0) & (cy2 - cy1 >= 1.0)

            @pl.when(have & nontiny)
            def _():
                row = jnp.where(iota == 0, cx1,
                                jnp.where(iota == 1, cy1,
                                          jnp.where(iota == 2, cx2,
                                                    jnp.where(iota == 3, cy2,
                                                              m))))
                out_v[pl.ds(si[0] * L, L)] = row
                si[0] = si[0] + np.int32(1)

            si[1] = jnp.where(have & (si[0] < TOPK),
                              np.int32(1), np.int32(0))
            si[2] = sel
            sf[0] = cx1
            sf[1] = cy1
            sf[2] = cx2
            sf[3] = cy2
            sf[4] = car

        nms_step(first=True)

        def outer(o, _):
            @pl.when(si[1] == np.int32(1))
            def _():
                def inner(k, __):
                    @pl.when(si[1] == np.int32(1))
                    def _():
                        nms_step(first=False)
                    return 0
                lax.fori_loop(0, SUP, inner, 0)
            return 0
        lax.fori_loop(0, NO, outer, 0)

        pltpu.sync_copy(out_v, out_hbm.at[b])


@jax.jit
def _nms_call(bt, sp):
    mesh = plsc.VectorSubcoreMesh(core_axis_name="c", subcore_axis_name="s")
    f = pl.kernel(
        _nms_body,
        out_type=jax.ShapeDtypeStruct((B, OUTW), jnp.float32),
        mesh=mesh,
        scratch_types=[
            pltpu.VMEM((NPAD + L,), jnp.float32),
            pltpu.VMEM((NPAD + L,), jnp.float32),
            pltpu.VMEM((NPAD + L,), jnp.float32),
            pltpu.VMEM((NPAD + L,), jnp.float32),
            pltpu.VMEM((NPAD,), jnp.float32),
            pltpu.VMEM((NPAD + L,), jnp.float32),
            pltpu.VMEM((OUTW,), jnp.float32),
            pltpu.SMEM((4,), jnp.int32),
            pltpu.SMEM((8,), jnp.float32),
        ],
    )
    return f(bt, sp)


def kernel(boxes, scores):
    bt = jnp.transpose(boxes, (0, 2, 1))
    bt = jnp.pad(bt, ((0, 0), (0, 0), (0, NPAD - N)))
    sp = jnp.pad(scores, ((0, 0), (0, NPAD - N)))
    out = _nms_call(bt, sp)
    return out[:, :TOPK * L].reshape(B, TOPK, L)[:, :, :TOPK]


# speedup vs baseline: 43807.1194x; 1.8480x over previous
import jax
import jax.numpy as jnp
import numpy as np
from jax import lax
from jax.experimental import pallas as pl
from jax.experimental.pallas import tpu as pltpu
from jax.experimental.pallas import tpu_sc as plsc

B = 4
N = 5000
TOPK = 5
L = 16
NPAD = 5120
NSH = 8
SEG = NPAD // NSH
LCH = SEG // L
NCORES = 2
NSUB = 16
OUTW = 128
SUP = 64
NO = 80
UNROLL = 4
CONF = np.float32(0.2)
IOU = np.float32(0.4)
NEG = np.float32(-1e30)
NEGHALF = np.float32(-5e29)
EPS = np.float32(1e-9)
BIGI = np.int32(2**30)
ONE = np.float32(1.0)
ZERO = np.float32(0.0)


def _nms_body(bt_hbm, sp_hbm, out_hbm,
              x1_v, y1_v, x2_v, y2_v, ws_v, area_v, out_v,
              rec_v, all_v, shared, si, sf):
    cid = lax.axis_index("c")
    sid = lax.axis_index("s")
    grp = sid // NSH
    lid = sid % NSH
    b = cid * 2 + grp
    base = lid * SEG
    gbase = grp * NSH

    pltpu.sync_copy(bt_hbm.at[b, 0, pl.ds(base, SEG)],
                    x1_v.at[pl.ds(0, SEG)])
    pltpu.sync_copy(bt_hbm.at[b, 1, pl.ds(base, SEG)],
                    y1_v.at[pl.ds(0, SEG)])
    pltpu.sync_copy(bt_hbm.at[b, 2, pl.ds(base, SEG)],
                    x2_v.at[pl.ds(0, SEG)])
    pltpu.sync_copy(bt_hbm.at[b, 3, pl.ds(base, SEG)],
                    y2_v.at[pl.ds(0, SEG)])
    pltpu.sync_copy(sp_hbm.at[b, pl.ds(base, SEG)], ws_v)

    iota = lax.iota(jnp.int32, L)
    zero16 = jnp.zeros((L,), jnp.float32)

    def share(rec):
        rec_v[pl.ds(0, L)] = rec
        pltpu.sync_copy(rec_v, shared.at[pl.ds(sid * L, L)])
        plsc.subcore_barrier()
        pltpu.sync_copy(shared, all_v)
        plsc.subcore_barrier()

    @plsc.parallel_loop(0, LCH, step=1, unroll=UNROLL,
                        carry=jnp.full((L,), NEG, jnp.float32))
    def m16(i, m):
        sl = pl.ds(i * L, L)
        area_v[sl] = (x2_v[sl] - x1_v[sl]) * (y2_v[sl] - y1_v[sl])
        return jnp.maximum(m, ws_v[sl])
    mr = m16
    for s in (8, 4, 2, 1):
        mr = jnp.maximum(mr, mr[iota ^ s])
    share(jnp.where(iota == 0, mr[0], ZERO))
    gmax = NEG
    for r in range(NSH):
        gmax = jnp.maximum(gmax, all_v[pl.ds((gbase + r) * L, L)][0])
    thr = jnp.where(gmax > CONF, CONF, ZERO)

    @pl.when(lid == 0)
    def _():
        def pz(r, _):
            out_v[pl.ds(r * L, L)] = zero16
            return 0
        lax.fori_loop(0, OUTW // L, pz, 0)

    si[0] = np.int32(0)
    si[1] = np.int32(1)
    si[2] = np.int32(-1)

    def nms_round(first):
        if first:
            prev_idx = np.int32(-1)
            nch = np.int32(LCH)
        else:
            prev_idx = si[2]
            px1 = sf[0]
            py1 = sf[1]
            px2 = sf[2]
            py2 = sf[3]
            par = sf[4]
            nch = jnp.where(si[1] == 1, np.int32(LCH), np.int32(0))

        @plsc.parallel_loop(
            0, nch, step=1, unroll=UNROLL,
            carry=(jnp.full((L,), NEG, jnp.float32),
                   jnp.full((L,), BIGI)))
        def final(i, carry):
            bv, bi = carry
            sl = pl.ds(i * L, L)
            ws = ws_v[sl]
            idxv = iota + (base + i * L)
            if first:
                wsn = jnp.where(ws > thr, ws, NEG)
            else:
                x1 = x1_v[sl]
                y1 = y1_v[sl]
                x2 = x2_v[sl]
                y2 = y2_v[sl]
                ar = area_v[sl]
                ix1 = jnp.maximum(px1, x1)
                iy1 = jnp.maximum(py1, y1)
                ix2 = jnp.minimum(px2, x2)
                iy2 = jnp.minimum(py2, y2)
                inter = (jnp.maximum(ix2 - ix1, 0.0)
                         * jnp.maximum(iy2 - iy1, 0.0))
                iou = inter / (par + ar - inter + EPS)
                sup = (iou > IOU) | (idxv == prev_idx)
                wsn = jnp.where(sup, NEG, ws)
            ws_v[sl] = wsn
            better = (wsn > bv) | ((wsn == bv) & (idxv < bi))
            bv = jnp.where(better, wsn, bv)
            bi = jnp.where(better, idxv, bi)
            return bv, bi

        bv, bi = final
        for s in (8, 4, 2, 1):
            gv = bv[iota ^ s]
            gi = bi[iota ^ s]
            better = (gv > bv) | ((gv == bv) & (gi < bi))
            bv = jnp.where(better, gv, bv)
            bi = jnp.where(better, gi, bi)
        lm = bv[0]
        lsel = bi[0]
        loff = jnp.where(lm > NEGHALF, lsel - base, np.int32(0))
        lx1 = x1_v[pl.ds(loff, L)][0]
        ly1 = y1_v[pl.ds(loff, L)][0]
        lx2 = x2_v[pl.ds(loff, L)][0]
        ly2 = y2_v[pl.ds(loff, L)][0]
        lar = area_v[pl.ds(loff, L)][0]
        contf = jnp.where(si[1] == 1, ONE, ZERO)

        rec = jnp.where(iota == 0, lm,
              jnp.where(iota == 1, lsel.astype(jnp.float32),
              jnp.where(iota == 2, lx1,
              jnp.where(iota == 3, ly1,
              jnp.where(iota == 4, lx2,
              jnp.where(iota == 5, ly2,
              jnp.where(iota == 6, lar,
              jnp.where(iota == 7, contf, ZERO))))))))
        share(rec)

        bm = NEG
        bidx = np.float32(2**30)
        brec = zero16
        for r in range(NSH):
            rr = all_v[pl.ds((gbase + r) * L, L)]
            rm = rr[0]
            ridx = rr[1]
            better = (rm > bm) | ((rm == bm) & (ridx < bidx))
            bm = jnp.where(better, rm, bm)
            bidx = jnp.where(better, ridx, bidx)
            mi = jnp.full((L,), jnp.where(better, np.int32(-1), np.int32(0)))
            rr_i = lax.bitcast_convert_type(rr, jnp.int32)
            br_i = lax.bitcast_convert_type(brec, jnp.int32)
            brec = lax.bitcast_convert_type((rr_i & mi) | (br_i & ~mi),
                                            jnp.float32)

        have = bm > NEGHALF
        cx1 = brec[2]
        cy1 = brec[3]
        cx2 = brec[4]
        cy2 = brec[5]
        nontiny = (cx2 - cx1 >= 1.0) & (cy2 - cy1 >= 1.0)
        emit = have & nontiny

        @pl.when(emit & (lid == 0))
        def _():
            perm = jnp.where(iota < 4, iota + 2,
                             jnp.where(iota == 4, 0, 7))
            out_v[pl.ds(si[0] * L, L)] = brec[perm]

        si[0] = si[0] + jnp.where(emit, np.int32(1), np.int32(0))
        si[1] = jnp.where(have & (si[0] < TOPK), np.int32(1), np.int32(0))
        si[2] = bidx.astype(jnp.int32)
        sf[0] = cx1
        sf[1] = cy1
        sf[2] = cx2
        sf[3] = cy2
        sf[4] = brec[6]

        anyc = ZERO
        for r in range(NCORES * NSH):
            anyc = jnp.maximum(anyc, all_v[pl.ds(r * L, L)][7])
        si[3] = jnp.where(anyc > 0.5, np.int32(1), np.int32(0))

    nms_round(first=True)

    def outer(o, _):
        @pl.when(si[3] == np.int32(1))
        def _():
            def inner(k, __):
                @pl.when(si[3] == np.int32(1))
                def _():
                    nms_round(first=False)
                return 0
            lax.fori_loop(0, SUP, inner, 0)
        return 0
    lax.fori_loop(0, NO, outer, 0)

    @pl.when(lid == 0)
    def _():
        pltpu.sync_copy(out_v, out_hbm.at[b])


@jax.jit
def _nms_call(bt, sp):
    mesh = plsc.VectorSubcoreMesh(core_axis_name="c", subcore_axis_name="s")
    f = pl.kernel(
        _nms_body,
        out_type=jax.ShapeDtypeStruct((B, OUTW), jnp.float32),
        mesh=mesh,
        scratch_types=[
            pltpu.VMEM((SEG + L,), jnp.float32),
            pltpu.VMEM((SEG + L,), jnp.float32),
            pltpu.VMEM((SEG + L,), jnp.float32),
            pltpu.VMEM((SEG + L,), jnp.float32),
            pltpu.VMEM((SEG,), jnp.float32),
            pltpu.VMEM((SEG + L,), jnp.float32),
            pltpu.VMEM((OUTW,), jnp.float32),
            pltpu.VMEM((L,), jnp.float32),
            pltpu.VMEM((NCORES * NSH * L,), jnp.float32),
            pltpu.VMEM_SHARED((NSUB * L,), jnp.float32),
            pltpu.SMEM((8,), jnp.int32),
            pltpu.SMEM((8,), jnp.float32),
        ],
    )
    return f(bt, sp)


def kernel(boxes, scores):
    bt = jnp.transpose(boxes, (0, 2, 1))
    bt = jnp.pad(bt, ((0, 0), (0, 0), (0, NPAD - N)))
    sp = jnp.pad(scores, ((0, 0), (0, NPAD - N)))
    out = _nms_call(bt, sp)
    return out[:, :TOPK * L].reshape(B, TOPK, L)[:, :, :TOPK]


# speedup vs baseline: 46184.1656x; 1.0543x over previous
import jax
import jax.numpy as jnp
import numpy as np
from jax import lax
from jax.experimental import pallas as pl
from jax.experimental.pallas import tpu as pltpu
from jax.experimental.pallas import tpu_sc as plsc

B = 4
N = 5000
TOPK = 5
L = 16
NPAD = 5120
NSH = 8
SEG = NPAD // NSH
LCH = SEG // L
NCORES = 2
NSUB = 16
RECW = NSUB * L
OUTW = 128
SUP = 64
NO = 80
UNROLL = 4
CONF = np.float32(0.2)
IOU = np.float32(0.4)
NEG = np.float32(-1e30)
NEGHALF = np.float32(-5e29)
EPS = np.float32(1e-9)
BIGI = np.int32(2**30)
ONE = np.float32(1.0)
ZERO = np.float32(0.0)


def _nms_body(bt_hbm, sp_hbm, out_hbm,
              x1_v, y1_v, x2_v, y2_v, ws_v, area_v, out_v,
              rec_v, all_v, shared, si, sf):
    cid = lax.axis_index("c")
    sid = lax.axis_index("s")
    grp = sid // NSH
    lid = sid % NSH
    b = cid * 2 + grp
    base = lid * SEG
    gbase = grp * NSH

    pltpu.sync_copy(bt_hbm.at[b, 0, pl.ds(base, SEG)],
                    x1_v.at[pl.ds(0, SEG)])
    pltpu.sync_copy(bt_hbm.at[b, 1, pl.ds(base, SEG)],
                    y1_v.at[pl.ds(0, SEG)])
    pltpu.sync_copy(bt_hbm.at[b, 2, pl.ds(base, SEG)],
                    x2_v.at[pl.ds(0, SEG)])
    pltpu.sync_copy(bt_hbm.at[b, 3, pl.ds(base, SEG)],
                    y2_v.at[pl.ds(0, SEG)])
    pltpu.sync_copy(sp_hbm.at[b, pl.ds(base, SEG)], ws_v)

    iota = lax.iota(jnp.int32, L)
    zero16 = jnp.zeros((L,), jnp.float32)

    @pl.when(lid == 0)
    def _():
        def pz(r, _):
            out_v[pl.ds(r * L, L)] = zero16
            return 0
        lax.fori_loop(0, OUTW // L, pz, 0)

    si[0] = np.int32(0)
    si[1] = np.int32(1)
    si[2] = np.int32(-1)
    si[4] = np.int32(0)

    def share(rec):
        par = si[4] & np.int32(1)
        off = par * RECW
        rec_v[pl.ds(0, L)] = rec
        pltpu.sync_copy(rec_v, shared.at[pl.ds(off + sid * L, L)])
        plsc.subcore_barrier()
        pltpu.sync_copy(shared.at[pl.ds(off, RECW)], all_v)
        si[4] = si[4] + np.int32(1)

    def nms_round(first):
        if first:
            prev_idx = np.int32(-1)
            nch = np.int32(LCH)
        else:
            prev_idx = si[2]
            px1 = sf[0]
            py1 = sf[1]
            px2 = sf[2]
            py2 = sf[3]
            par = sf[4]
            nch = jnp.where(si[1] == 1, np.int32(LCH), np.int32(0))

        @plsc.parallel_loop(
            0, nch, step=1, unroll=UNROLL,
            carry=(jnp.full((L,), NEG, jnp.float32),
                   jnp.full((L,), BIGI)))
        def final(i, carry):
            bv, bi = carry
            sl = pl.ds(i * L, L)
            ws = ws_v[sl]
            idxv = iota + (base + i * L)
            if first:
                area_v[sl] = (x2_v[sl] - x1_v[sl]) * (y2_v[sl] - y1_v[sl])
                wsn = ws
            else:
                x1 = x1_v[sl]
                y1 = y1_v[sl]
                x2 = x2_v[sl]
                y2 = y2_v[sl]
                ar = area_v[sl]
                ix1 = jnp.maximum(px1, x1)
                iy1 = jnp.maximum(py1, y1)
                ix2 = jnp.minimum(px2, x2)
                iy2 = jnp.minimum(py2, y2)
                inter = (jnp.maximum(ix2 - ix1, 0.0)
                         * jnp.maximum(iy2 - iy1, 0.0))
                iou = inter / (par + ar - inter + EPS)
                sup = (iou > IOU) | (idxv == prev_idx)
                wsn = jnp.where(sup, NEG, ws)
                ws_v[sl] = wsn
            better = (wsn > bv) | ((wsn == bv) & (idxv < bi))
            bv = jnp.where(better, wsn, bv)
            bi = jnp.where(better, idxv, bi)
            return bv, bi

        bv, bi = final
        for s in (8, 4, 2, 1):
            gv = bv[iota ^ s]
            gi = bi[iota ^ s]
            better = (gv > bv) | ((gv == bv) & (gi < bi))
            bv = jnp.where(better, gv, bv)
            bi = jnp.where(better, gi, bi)
        lm = bv[0]
        lsel = bi[0]
        loff = jnp.where(lm > NEGHALF, lsel - base, np.int32(0))
        lx1 = x1_v[pl.ds(loff, L)][0]
        ly1 = y1_v[pl.ds(loff, L)][0]
        lx2 = x2_v[pl.ds(loff, L)][0]
        ly2 = y2_v[pl.ds(loff, L)][0]
        lar = area_v[pl.ds(loff, L)][0]
        contf = jnp.where(si[1] == 1, ONE, ZERO)

        rec = jnp.where(iota == 0, lm,
              jnp.where(iota == 1, lsel.astype(jnp.float32),
              jnp.where(iota == 2, lx1,
              jnp.where(iota == 3, ly1,
              jnp.where(iota == 4, lx2,
              jnp.where(iota == 5, ly2,
              jnp.where(iota == 6, lar,
              jnp.where(iota == 7, contf, ZERO))))))))
        share(rec)

        bm = NEG
        bidx = np.float32(2**30)
        brec = zero16
        for r in range(NSH):
            rr = all_v[pl.ds((gbase + r) * L, L)]
            rm = rr[0]
            ridx = rr[1]
            better = (rm > bm) | ((rm == bm) & (ridx < bidx))
            bm = jnp.where(better, rm, bm)
            bidx = jnp.where(better, ridx, bidx)
            mi = jnp.full((L,), jnp.where(better, np.int32(-1), np.int32(0)))
            rr_i = lax.bitcast_convert_type(rr, jnp.int32)
            br_i = lax.bitcast_convert_type(brec, jnp.int32)
            brec = lax.bitcast_convert_type((rr_i & mi) | (br_i & ~mi),
                                            jnp.float32)
        acc = zero16
        for r in range(NCORES * NSH):
            acc = jnp.maximum(acc, all_v[pl.ds(r * L, L)])
        anyc = acc[7]

        if first:
            thr = jnp.where(bm > CONF, CONF, ZERO)
            sf[5] = thr
        else:
            thr = sf[5]
        have = bm > thr
        cx1 = brec[2]
        cy1 = brec[3]
        cx2 = brec[4]
        cy2 = brec[5]
        nontiny = (cx2 - cx1 >= 1.0) & (cy2 - cy1 >= 1.0)
        emit = have & nontiny

        @pl.when(emit & (lid == 0))
        def _():
            perm = jnp.where(iota < 4, iota + 2,
                             jnp.where(iota == 4, 0, 7))
            out_v[pl.ds(si[0] * L, L)] = brec[perm]

        si[0] = si[0] + jnp.where(emit, np.int32(1), np.int32(0))
        si[1] = jnp.where(have & (si[0] < TOPK), np.int32(1), np.int32(0))
        si[2] = bidx.astype(jnp.int32)
        sf[0] = cx1
        sf[1] = cy1
        sf[2] = cx2
        sf[3] = cy2
        sf[4] = brec[6]

        si[3] = jnp.where(anyc > 0.5, np.int32(1), np.int32(0))

    nms_round(first=True)

    def outer(o, _):
        @pl.when(si[3] == np.int32(1))
        def _():
            def inner(k, __):
                @pl.when(si[3] == np.int32(1))
                def _():
                    nms_round(first=False)
                return 0
            lax.fori_loop(0, SUP, inner, 0)
        return 0
    lax.fori_loop(0, NO, outer, 0)

    @pl.when(lid == 0)
    def _():
        pltpu.sync_copy(out_v, out_hbm.at[b])


@jax.jit
def _nms_call(bt, sp):
    mesh = plsc.VectorSubcoreMesh(core_axis_name="c", subcore_axis_name="s")
    f = pl.kernel(
        _nms_body,
        out_type=jax.ShapeDtypeStruct((B, OUTW), jnp.float32),
        mesh=mesh,
        scratch_types=[
            pltpu.VMEM((SEG + L,), jnp.float32),
            pltpu.VMEM((SEG + L,), jnp.float32),
            pltpu.VMEM((SEG + L,), jnp.float32),
            pltpu.VMEM((SEG + L,), jnp.float32),
            pltpu.VMEM((SEG,), jnp.float32),
            pltpu.VMEM((SEG + L,), jnp.float32),
            pltpu.VMEM((OUTW,), jnp.float32),
            pltpu.VMEM((L,), jnp.float32),
            pltpu.VMEM((RECW,), jnp.float32),
            pltpu.VMEM_SHARED((2 * RECW,), jnp.float32),
            pltpu.SMEM((8,), jnp.int32),
            pltpu.SMEM((8,), jnp.float32),
        ],
    )
    return f(bt, sp)


def kernel(boxes, scores):
    bt = jnp.transpose(boxes, (0, 2, 1))
    bt = jnp.pad(bt, ((0, 0), (0, 0), (0, NPAD - N)))
    sp = jnp.pad(scores, ((0, 0), (0, NPAD - N)))
    out = _nms_call(bt, sp)
    return out[:, :TOPK * L].reshape(B, TOPK, L)[:, :, :TOPK]
